# baseline (speedup 1.0000x reference)
import functools

import jax
import jax.numpy as jnp
from jax import lax
from jax.experimental import pallas as pl
from jax.experimental.pallas import tpu as pltpu
from jax.experimental.pallas import tpu_sc as plsc

B = 4096
S = 200
D = 32
NC = 2
NS = 16
NW = NC * NS
TOTAL = B * S
PER_W = TOTAL // NW
R = 800
NCH = PER_W // R
REPS = R // S

_mesh = plsc.VectorSubcoreMesh(core_axis_name="c", subcore_axis_name="s")


@functools.partial(
    pl.kernel,
    mesh=_mesh,
    compiler_params=pltpu.CompilerParams(use_tc_tiling_on_sc=False),
    out_type=jax.ShapeDtypeStruct((TOTAL, D), jnp.float32),
    scratch_types=[
        pltpu.VMEM((PER_W,), jnp.int32),
        pltpu.VMEM((R, D), jnp.float32),
        pltpu.VMEM((R, D), jnp.float32),
        pltpu.SemaphoreType.DMA,
        pltpu.SemaphoreType.DMA,
        pltpu.SemaphoreType.DMA,
        pltpu.SemaphoreType.DMA,
        pltpu.SemaphoreType.DMA,
        pltpu.SemaphoreType.DMA,
    ],
)
def _embed(x_hbm, tok_hbm, pos_rep_hbm, out_hbm,
           idx_v, rows0, rows1,
           psem0, psem1, gsem0, gsem1, ssem0, ssem1):
    wid = lax.axis_index("s") * NC + lax.axis_index("c")
    base = wid * PER_W

    rows = (rows0, rows1)
    psem = (psem0, psem1)
    gsem = (gsem0, gsem1)
    ssem = (ssem0, ssem1)

    pltpu.sync_copy(x_hbm.at[pl.ds(base, PER_W)], idx_v)

    gd = [None, None]
    sd = [None, None]
    for c in range(NCH):
        buf = c % 2
        oth = 1 - buf
        if sd[buf] is not None:
            sd[buf].wait()
        pd = pltpu.async_copy(pos_rep_hbm, rows[buf], psem[buf])
        if gd[oth] is not None:
            gd[oth].wait()
            sd[oth] = pltpu.async_copy(
                rows[oth], out_hbm.at[pl.ds(base + (c - 1) * R, R)],
                ssem[oth])
        pd.wait()
        gd[buf] = pltpu.async_copy(
            tok_hbm.at[idx_v.at[pl.ds(c * R, R)]], rows[buf], gsem[buf],
            add=True)
    last = (NCH - 1) % 2
    gd[last].wait()
    sd[last] = pltpu.async_copy(
        rows[last], out_hbm.at[pl.ds(base + (NCH - 1) * R, R)], ssem[last])
    sd[1 - last].wait()
    sd[last].wait()


def kernel(x, token_table, pos_table):
    xf = x.reshape(-1).astype(jnp.int32)
    pos_rep = jnp.tile(pos_table, (REPS, 1))
    out = _embed(xf, token_table, pos_rep)
    return out.reshape(B, S, D)


# speedup vs baseline: 1.1557x; 1.1557x over previous
import functools

import jax
import jax.numpy as jnp
from jax import lax
from jax.experimental import pallas as pl
from jax.experimental.pallas import tpu as pltpu
from jax.experimental.pallas import tpu_sc as plsc

B = 4096
S = 200
D = 32
NC = 2
NS = 16
NW = NC * NS
TOTAL = B * S
PER_W = TOTAL // NW
SEQ_W = PER_W // S
R = 800
SEQ_C = R // S
NCH = PER_W // R

_mesh = plsc.VectorSubcoreMesh(core_axis_name="c", subcore_axis_name="s")


@functools.partial(
    pl.kernel,
    mesh=_mesh,
    compiler_params=pltpu.CompilerParams(use_tc_tiling_on_sc=False),
    out_type=jax.ShapeDtypeStruct((B, S, D), jnp.float32),
    scratch_types=[
        pltpu.VMEM((PER_W,), jnp.int32),
        pltpu.VMEM((R, D), jnp.float32),
        pltpu.VMEM((R, D), jnp.float32),
        pltpu.VMEM((S, D), jnp.float32),
        pltpu.SemaphoreType.DMA,
        pltpu.SemaphoreType.DMA,
        pltpu.SemaphoreType.DMA,
        pltpu.SemaphoreType.DMA,
    ],
)
def _embed(x_hbm, tok_hbm, pos_hbm, out_hbm,
           idx_v, rows0, rows1, pos_v,
           gsem0, gsem1, ssem0, ssem1):
    wid = lax.axis_index("s") * NC + lax.axis_index("c")
    base = wid * PER_W
    seq_base = wid * SEQ_W

    rows = (rows0, rows1)
    gsem = (gsem0, gsem1)
    ssem = (ssem0, ssem1)

    pltpu.sync_copy(x_hbm.at[pl.ds(base, PER_W)], idx_v)
    pltpu.sync_copy(pos_hbm, pos_v)

    def start_gather(c):
        buf = c % 2
        return pltpu.async_copy(
            tok_hbm.at[idx_v.at[pl.ds(c * R, R)]], rows[buf], gsem[buf])

    def add_pos(buf):
        rv = rows[buf]

        def body(p, _):
            lo = pos_v[p, pl.ds(0, 16)]
            hi = pos_v[p, pl.ds(16, 16)]
            for k in range(SEQ_C):
                r = k * S + p
                rv[r, pl.ds(0, 16)] = rv[r, pl.ds(0, 16)] + lo
                rv[r, pl.ds(16, 16)] = rv[r, pl.ds(16, 16)] + hi
            return 0

        lax.fori_loop(0, S, body, 0)

    def start_scatter(c):
        buf = c % 2
        descs = []
        for k in range(SEQ_C):
            descs.append(pltpu.async_copy(
                rows[buf].at[pl.ds(k * S, S)],
                out_hbm.at[seq_base + c * SEQ_C + k],
                ssem[buf]))
        return descs

    def wait_all(descs):
        for d in descs:
            d.wait()

    gd = [None, None]
    sd = [None, None]
    gd[0] = start_gather(0)
    for c in range(NCH):
        buf = c % 2
        oth = 1 - buf
        if c + 1 < NCH:
            if sd[oth] is not None:
                wait_all(sd[oth])
            gd[oth] = start_gather(c + 1)
        gd[buf].wait()
        add_pos(buf)
        sd[buf] = start_scatter(c)
    wait_all(sd[0])
    wait_all(sd[1])


def kernel(x, token_table, pos_table):
    xf = x.reshape(-1).astype(jnp.int32)
    return _embed(xf, token_table, pos_table)
